# baseline (speedup 1.0000x reference)
import functools

import jax
import jax.numpy as jnp
from jax import lax
from jax.experimental import pallas as pl
from jax.experimental.pallas import tpu as pltpu, tpu_sc as plsc

P = 3
OUT_DIM = 4096
N_CTRL = 8192
N_KNOTS = N_CTRL + P + 1
N_UP = N_KNOTS - P

NC, NS, L = 2, 16, 16
NW = NC * NS
S_PER_W = OUT_DIM // NW
V_PER_W = S_PER_W // L

U_START = 1e-5
U_STEP = (1.0 - 2e-5) / (OUT_DIM - 1)
EPS = 1e-8
DEG_EPS = 1e-4


def _basis_step(Nr, U1, U2, u, saved):
    dU = (U1 - u) + (u - U2)
    zero = dU == 0.0
    dU_ = jnp.where(zero, DEG_EPS, dU)
    temp = Nr / dU_
    temp = jnp.where(zero, DEG_EPS, temp)
    return saved + (U1 - u) * temp, (u - U2) * temp


def _sc_body(ctrl_hbm, knots_hbm, out_hbm, knots_v, ctrl_v, outbuf,
             sem_k, sem_c, sem_o):
    wid = lax.axis_index("s") * NC + lax.axis_index("c")
    cp_k = pltpu.make_async_copy(knots_hbm.at[0], knots_v, sem_k)
    cp_c = pltpu.make_async_copy(ctrl_hbm.at[0], ctrl_v, sem_c)
    cp_k.start()
    cp_c.start()

    lanes = lax.iota(jnp.int32, L)
    zeros_i = jnp.zeros((L,), jnp.int32)

    def gk(idx):
        safe = jnp.minimum(jnp.maximum(idx, 0), N_KNOTS - 1)
        return plsc.load_gather(knots_v, [safe])

    cp_k.wait()

    us = [U_START + (wid * S_PER_W + v * L + lanes).astype(jnp.float32) * U_STEP
          for v in range(V_PER_W)]
    ms = [zeros_i for _ in range(V_PER_W)]
    step = 8192
    while step >= 1:
        for v in range(V_PER_W):
            cand = ms[v] + step
            valid = cand <= N_UP
            j = jnp.minimum(cand, N_UP) - 1
            vvals = gk(j + P)
            pred = valid & ((us[v] - vvals) > EPS)
            ms[v] = jnp.where(pred, cand, ms[v])
        step //= 2

    offs, weights = [], []
    for v in range(V_PER_W):
        u, m = us[v], ms[v]
        d = u - gk(jnp.maximum(m, 1) + P - 1)
        off = jnp.where(m == 0, zeros_i,
                        jnp.where((d > 1.0) & (m < N_UP), m, m - 1))
        uspan = off + P

        A1, A2, A3 = gk(uspan + 1), gk(uspan + 2), gk(uspan + 3)
        B0, B1, B2 = gk(uspan), gk(uspan - 1), gk(uspan - 2)

        zero = jnp.zeros((L,), jnp.float32)
        N0 = jnp.ones((L,), jnp.float32)
        N0, s = _basis_step(N0, A1, B0, u, zero)
        N1 = s
        N0, s = _basis_step(N0, A1, B1, u, zero)
        N1, s = _basis_step(N1, A2, B0, u, s)
        N2 = s
        N0, s = _basis_step(N0, A1, B2, u, zero)
        N1, s = _basis_step(N1, A2, B1, u, s)
        N2, s = _basis_step(N2, A3, B0, u, s)
        N3 = s
        offs.append(off)
        weights.append((N0, N1, N2, N3))

    cp_c.wait()

    for v in range(V_PER_W):
        off = offs[v]
        N0, N1, N2, N3 = weights[v]
        rs = [jnp.minimum(off + j, N_CTRL - 1) for j in range(4)]
        pos = v * L + lanes
        for dim in range(3):
            dsp = jnp.full((L,), dim, jnp.int32)
            c0 = plsc.load_gather(ctrl_v, [rs[0], dsp])
            c1 = plsc.load_gather(ctrl_v, [rs[1], dsp])
            c2 = plsc.load_gather(ctrl_v, [rs[2], dsp])
            c3 = plsc.load_gather(ctrl_v, [rs[3], dsp])
            val = ((N0 * c0 + N1 * c1) + N2 * c2) + N3 * c3
            plsc.store_scatter(outbuf, [pos, dsp], val)

    pltpu.make_async_copy(
        outbuf, out_hbm.at[0, pl.ds(wid * S_PER_W, S_PER_W)], sem_o
    ).start()
    pltpu.make_async_copy(
        outbuf, out_hbm.at[0, pl.ds(wid * S_PER_W, S_PER_W)], sem_o
    ).wait()


@jax.jit
def _launch(ctrl_pts, knot_u):
    mesh = plsc.VectorSubcoreMesh(core_axis_name="c", subcore_axis_name="s")
    run = functools.partial(
        pl.kernel,
        mesh=mesh,
        out_type=jax.ShapeDtypeStruct((1, OUT_DIM, 3), jnp.float32),
        scratch_types=[
            pltpu.VMEM((N_KNOTS,), jnp.float32),
            pltpu.VMEM((N_CTRL, 3), jnp.float32),
            pltpu.VMEM((S_PER_W, 3), jnp.float32),
            pltpu.SemaphoreType.DMA,
            pltpu.SemaphoreType.DMA,
            pltpu.SemaphoreType.DMA,
        ],
        compiler_params=pltpu.CompilerParams(
            needs_layout_passes=False, use_tc_tiling_on_sc=False
        ),
    )(_sc_body)
    return run(ctrl_pts, knot_u)


def kernel(ctrl_pts, knot_u):
    return _launch(ctrl_pts, knot_u)
